# baseline (device time: 20174 ns/iter reference)
import jax
import jax.numpy as jnp
from jax import lax
from jax.experimental import pallas as pl
from jax.experimental.pallas import tpu as pltpu

N_DEV = 8
B = 2
SQ = 128
HQ_LOCAL = 4
DH = 64
HD_LOCAL = HQ_LOCAL * DH
D_MODEL = 512
ROWS = B * SQ
CHUNK = ROWS // N_DEV

bf16 = jnp.bfloat16
f32 = jnp.float32


def kernel(x, Wq, K_ext, V_ext, Wo):
    def body(x_ref, wq_ref, kv_ref, wo_ref, out_ref,
             ctx_ref, part_ref, red_ref, red16_ref, rs_ref, ag_ref,
             rs_send_sems, rs_recv_sems, ag_send_sems, ag_recv_sems):
        my = lax.axis_index("i")

        barrier_sem = pltpu.get_barrier_semaphore()
        for o in range(1, N_DEV):
            pl.semaphore_signal(barrier_sem, inc=1,
                                device_id=(lax.rem(my + o, N_DEV),),
                                device_id_type=pl.DeviceIdType.MESH)

        q = jnp.dot(x_ref[...].astype(bf16), wq_ref[...].astype(bf16),
                    preferred_element_type=f32)
        q = (q * 0.125).astype(bf16)
        for b in range(B):
            for h in range(HQ_LOCAL):
                qh = q[b * SQ:(b + 1) * SQ, h * DH:(h + 1) * DH]
                kh = kv_ref[0, b * HQ_LOCAL + h]
                vh = kv_ref[1, b * HQ_LOCAL + h]
                scores = lax.dot_general(
                    qh, kh, (((1,), (1,)), ((), ())),
                    preferred_element_type=f32)
                w = jnp.exp(scores)
                s = jnp.sum(w, axis=-1, keepdims=True)
                ctx = jnp.dot(w.astype(bf16), vh,
                              preferred_element_type=f32)
                ctx_ref[b * SQ:(b + 1) * SQ, h * DH:(h + 1) * DH] = (
                    ctx * (1.0 / s)).astype(bf16)

        pl.semaphore_wait(barrier_sem, N_DEV - 1)

        def send_chunk(c):
            @pl.when(my != c)
            def _():
                rdma = pltpu.make_async_remote_copy(
                    src_ref=part_ref.at[pl.ds(c * CHUNK, CHUNK), :],
                    dst_ref=rs_ref.at[my],
                    send_sem=rs_send_sems.at[c],
                    recv_sem=rs_recv_sems.at[my],
                    device_id=(c,),
                    device_id_type=pl.DeviceIdType.MESH,
                )
                rdma.start()

        half = ROWS // 2
        wo16 = wo_ref[...].astype(bf16)
        part_ref[:half, :] = jnp.dot(
            ctx_ref[:half, :], wo16,
            preferred_element_type=f32).astype(bf16)
        for c in range(N_DEV // 2):
            send_chunk(c)
        part_ref[half:, :] = jnp.dot(
            ctx_ref[half:, :], wo16,
            preferred_element_type=f32).astype(bf16)
        for c in range(N_DEV // 2, N_DEV):
            send_chunk(c)

        red_ref[...] = part_ref[pl.ds(my * CHUNK, CHUNK), :].astype(f32)
        for s_ in range(N_DEV):
            @pl.when(my != s_)
            def _():
                recv = pltpu.make_async_remote_copy(
                    src_ref=rs_ref.at[s_], dst_ref=rs_ref.at[s_],
                    send_sem=rs_send_sems.at[s_],
                    recv_sem=rs_recv_sems.at[s_],
                    device_id=(s_,), device_id_type=pl.DeviceIdType.MESH,
                )
                recv.wait_recv()
                red_ref[...] = red_ref[...] + rs_ref[s_].astype(f32)

        red16_ref[...] = red_ref[...].astype(bf16)
        out_ref[pl.ds(my * CHUNK, CHUNK), :] = red_ref[...]
        for c in range(N_DEV):
            @pl.when(my != c)
            def _():
                rdma = pltpu.make_async_remote_copy(
                    src_ref=red16_ref,
                    dst_ref=ag_ref.at[my],
                    send_sem=ag_send_sems.at[c],
                    recv_sem=ag_recv_sems.at[my],
                    device_id=(c,),
                    device_id_type=pl.DeviceIdType.MESH,
                )
                rdma.start()
        for s_ in range(N_DEV):
            @pl.when(my != s_)
            def _():
                recv = pltpu.make_async_remote_copy(
                    src_ref=red16_ref,
                    dst_ref=ag_ref.at[s_],
                    send_sem=ag_send_sems.at[s_],
                    recv_sem=ag_recv_sems.at[s_],
                    device_id=(s_,), device_id_type=pl.DeviceIdType.MESH,
                )
                recv.wait_recv()
                out_ref[pl.ds(s_ * CHUNK, CHUNK), :] = ag_ref[s_].astype(f32)

        for c in range(N_DEV):
            @pl.when(my != c)
            def _():
                send = pltpu.make_async_remote_copy(
                    src_ref=part_ref.at[pl.ds(c * CHUNK, CHUNK), :],
                    dst_ref=rs_ref.at[my],
                    send_sem=rs_send_sems.at[c],
                    recv_sem=rs_recv_sems.at[my],
                    device_id=(c,), device_id_type=pl.DeviceIdType.MESH,
                )
                send.wait_send()
                send2 = pltpu.make_async_remote_copy(
                    src_ref=red16_ref,
                    dst_ref=ag_ref.at[my],
                    send_sem=ag_send_sems.at[c],
                    recv_sem=ag_recv_sems.at[my],
                    device_id=(c,), device_id_type=pl.DeviceIdType.MESH,
                )
                send2.wait_send()

    kv2 = (jnp.stack([K_ext, V_ext])
           .transpose(0, 1, 3, 2, 4)
           .reshape(2, B * HQ_LOCAL, SQ, DH)
           .astype(bf16))
    x2 = x.reshape(ROWS, -1)

    out2d = pl.pallas_call(
        body,
        grid=(1,),
        out_shape=jax.ShapeDtypeStruct((ROWS, D_MODEL), f32),
        in_specs=[
            pl.BlockSpec((ROWS, D_MODEL), lambda i: (0, 0),
                         memory_space=pltpu.VMEM),
            pl.BlockSpec((D_MODEL, HD_LOCAL),
                         lambda i: (0, lax.axis_index("i")),
                         memory_space=pltpu.VMEM),
            pl.BlockSpec((2, B * HQ_LOCAL, SQ, DH),
                         lambda i: (0, 0, 0, 0), memory_space=pltpu.VMEM),
            pl.BlockSpec((HD_LOCAL, D_MODEL),
                         lambda i: (lax.axis_index("i"), 0),
                         memory_space=pltpu.VMEM),
        ],
        out_specs=pl.BlockSpec((ROWS, D_MODEL), lambda i: (0, 0),
                               memory_space=pltpu.VMEM),
        scratch_shapes=[
            pltpu.VMEM((ROWS, HD_LOCAL), bf16),
            pltpu.VMEM((ROWS, D_MODEL), bf16),
            pltpu.VMEM((CHUNK, D_MODEL), f32),
            pltpu.VMEM((CHUNK, D_MODEL), bf16),
            pltpu.VMEM((N_DEV, CHUNK, D_MODEL), bf16),
            pltpu.VMEM((N_DEV, CHUNK, D_MODEL), bf16),
            pltpu.SemaphoreType.DMA((N_DEV,)),
            pltpu.SemaphoreType.DMA((N_DEV,)),
            pltpu.SemaphoreType.DMA((N_DEV,)),
            pltpu.SemaphoreType.DMA((N_DEV,)),
        ],
        compiler_params=pltpu.CompilerParams(collective_id=0),
    )(x2, Wq, kv2, Wo)
    return out2d.reshape(B, SQ, D_MODEL)


# device time: 17478 ns/iter; 1.1543x vs baseline; 1.1543x over previous
import jax
import jax.numpy as jnp
from jax import lax
from jax.experimental import pallas as pl
from jax.experimental.pallas import tpu as pltpu

N_DEV = 8
B = 2
SQ = 128
HQ_LOCAL = 4
DH = 64
HD_LOCAL = HQ_LOCAL * DH
D_MODEL = 512
ROWS = B * SQ
CHUNK = ROWS // N_DEV

bf16 = jnp.bfloat16
f32 = jnp.float32


def kernel(x, Wq, K_ext, V_ext, Wo):
    def body(x_ref, w_ref, kv_ref, out_ref,
             ctx_ref, part_ref, red_ref, red16_ref, rs_ref, ag_ref,
             rs_send_sems, rs_recv_sems, ag_send_sems, ag_recv_sems):
        my = lax.axis_index("i")

        barrier_sem = pltpu.get_barrier_semaphore()
        for o in range(1, N_DEV):
            pl.semaphore_signal(barrier_sem, inc=1,
                                device_id=(lax.rem(my + o, N_DEV),),
                                device_id_type=pl.DeviceIdType.MESH)

        q = jnp.dot(x_ref[...], w_ref[:, :HD_LOCAL],
                    preferred_element_type=f32).astype(bf16)
        for b in range(B):
            for h in range(HQ_LOCAL):
                qh = q[b * SQ:(b + 1) * SQ, h * DH:(h + 1) * DH]
                kh = kv_ref[0, b * HQ_LOCAL + h]
                vh = kv_ref[1, b * HQ_LOCAL + h]
                scores = lax.dot_general(
                    qh, kh, (((1,), (1,)), ((), ())),
                    preferred_element_type=f32)
                w = jnp.exp(scores)
                s = jnp.sum(w, axis=-1, keepdims=True)
                ctx = jnp.dot(w.astype(bf16), vh,
                              preferred_element_type=f32)
                ctx_ref[b * SQ:(b + 1) * SQ, h * DH:(h + 1) * DH] = (
                    ctx * (1.0 / s)).astype(bf16)

        pl.semaphore_wait(barrier_sem, N_DEV - 1)

        def send_chunk(c):
            @pl.when(my != c)
            def _():
                rdma = pltpu.make_async_remote_copy(
                    src_ref=part_ref.at[pl.ds(c * CHUNK, CHUNK), :],
                    dst_ref=rs_ref.at[my],
                    send_sem=rs_send_sems.at[c],
                    recv_sem=rs_recv_sems.at[my],
                    device_id=(c,),
                    device_id_type=pl.DeviceIdType.MESH,
                )
                rdma.start()

        half = ROWS // 2
        woT = w_ref[:, HD_LOCAL:]
        part_ref[:half, :] = lax.dot_general(
            ctx_ref[:half, :], woT, (((1,), (1,)), ((), ())),
            preferred_element_type=f32).astype(bf16)
        for c in range(N_DEV // 2):
            send_chunk(c)
        part_ref[half:, :] = lax.dot_general(
            ctx_ref[half:, :], woT, (((1,), (1,)), ((), ())),
            preferred_element_type=f32).astype(bf16)
        for c in range(N_DEV // 2, N_DEV):
            send_chunk(c)

        red_ref[...] = part_ref[pl.ds(my * CHUNK, CHUNK), :].astype(f32)
        for s_ in range(N_DEV):
            @pl.when(my != s_)
            def _():
                recv = pltpu.make_async_remote_copy(
                    src_ref=rs_ref.at[s_], dst_ref=rs_ref.at[s_],
                    send_sem=rs_send_sems.at[s_],
                    recv_sem=rs_recv_sems.at[s_],
                    device_id=(s_,), device_id_type=pl.DeviceIdType.MESH,
                )
                recv.wait_recv()
                red_ref[...] = red_ref[...] + rs_ref[s_].astype(f32)

        red16_ref[...] = red_ref[...].astype(bf16)
        out_ref[pl.ds(my * CHUNK, CHUNK), :] = red_ref[...]
        for c in range(N_DEV):
            @pl.when(my != c)
            def _():
                rdma = pltpu.make_async_remote_copy(
                    src_ref=red16_ref,
                    dst_ref=ag_ref.at[my],
                    send_sem=ag_send_sems.at[c],
                    recv_sem=ag_recv_sems.at[my],
                    device_id=(c,),
                    device_id_type=pl.DeviceIdType.MESH,
                )
                rdma.start()
        for s_ in range(N_DEV):
            @pl.when(my != s_)
            def _():
                recv = pltpu.make_async_remote_copy(
                    src_ref=red16_ref,
                    dst_ref=ag_ref.at[s_],
                    send_sem=ag_send_sems.at[s_],
                    recv_sem=ag_recv_sems.at[s_],
                    device_id=(s_,), device_id_type=pl.DeviceIdType.MESH,
                )
                recv.wait_recv()
                out_ref[pl.ds(s_ * CHUNK, CHUNK), :] = ag_ref[s_].astype(f32)

        for c in range(N_DEV):
            @pl.when(my != c)
            def _():
                send = pltpu.make_async_remote_copy(
                    src_ref=part_ref.at[pl.ds(c * CHUNK, CHUNK), :],
                    dst_ref=rs_ref.at[my],
                    send_sem=rs_send_sems.at[c],
                    recv_sem=rs_recv_sems.at[my],
                    device_id=(c,), device_id_type=pl.DeviceIdType.MESH,
                )
                send.wait_send()
                send2 = pltpu.make_async_remote_copy(
                    src_ref=red16_ref,
                    dst_ref=ag_ref.at[my],
                    send_sem=ag_send_sems.at[c],
                    recv_sem=ag_recv_sems.at[my],
                    device_id=(c,), device_id_type=pl.DeviceIdType.MESH,
                )
                send2.wait_send()

    my = lax.axis_index("i")
    wq_local = lax.dynamic_slice(
        Wq, (0, my * HD_LOCAL), (Wq.shape[0], HD_LOCAL))
    wo_local = lax.dynamic_slice(
        Wo, (my * HD_LOCAL, 0), (HD_LOCAL, Wo.shape[1]))
    w2 = jnp.concatenate([wq_local * 0.125, wo_local.T], axis=1).astype(bf16)
    kv2 = (jnp.stack([K_ext, V_ext])
           .transpose(0, 1, 3, 2, 4)
           .reshape(2, B * HQ_LOCAL, SQ, DH)
           .astype(bf16))
    x2 = x.reshape(ROWS, -1).astype(bf16)

    out2d = pl.pallas_call(
        body,
        out_shape=jax.ShapeDtypeStruct((ROWS, D_MODEL), f32),
        in_specs=[pl.BlockSpec(memory_space=pltpu.VMEM)] * 3,
        out_specs=pl.BlockSpec(memory_space=pltpu.VMEM),
        scratch_shapes=[
            pltpu.VMEM((ROWS, HD_LOCAL), bf16),
            pltpu.VMEM((ROWS, D_MODEL), bf16),
            pltpu.VMEM((CHUNK, D_MODEL), f32),
            pltpu.VMEM((CHUNK, D_MODEL), bf16),
            pltpu.VMEM((N_DEV, CHUNK, D_MODEL), bf16),
            pltpu.VMEM((N_DEV, CHUNK, D_MODEL), bf16),
            pltpu.SemaphoreType.DMA((N_DEV,)),
            pltpu.SemaphoreType.DMA((N_DEV,)),
            pltpu.SemaphoreType.DMA((N_DEV,)),
            pltpu.SemaphoreType.DMA((N_DEV,)),
        ],
        compiler_params=pltpu.CompilerParams(collective_id=0),
    )(x2, w2, kv2)
    return out2d.reshape(B, SQ, D_MODEL)


# device time: 17034 ns/iter; 1.1843x vs baseline; 1.0261x over previous
import jax
import jax.numpy as jnp
from jax import lax
from jax.experimental import pallas as pl
from jax.experimental.pallas import tpu as pltpu

N_DEV = 8
B = 2
SQ = 128
HQ_LOCAL = 4
DH = 64
HD_LOCAL = HQ_LOCAL * DH
D_MODEL = 512
ROWS = B * SQ
CHUNK = ROWS // N_DEV

bf16 = jnp.bfloat16
f32 = jnp.float32


def kernel(x, Wq, K_ext, V_ext, Wo):
    def body(x_ref, wq_ref, kv_ref, wo_ref, out_ref,
             ctx_ref, part_ref, red_ref, red16_ref, rs_ref, ag_ref,
             rs_send_sems, rs_recv_sems, ag_send_sems, ag_recv_sems):
        my = lax.axis_index("i")

        barrier_sem = pltpu.get_barrier_semaphore()
        for o in range(1, N_DEV):
            pl.semaphore_signal(barrier_sem, inc=1,
                                device_id=(lax.rem(my + o, N_DEV),),
                                device_id_type=pl.DeviceIdType.MESH)

        q = jnp.dot(x_ref[...].astype(bf16), wq_ref[...],
                    preferred_element_type=f32)
        q = (q * 0.125).astype(bf16)
        for b in range(B):
            for h in range(HQ_LOCAL):
                qh = q[b * SQ:(b + 1) * SQ, h * DH:(h + 1) * DH]
                kh = kv_ref[0, b * HQ_LOCAL + h]
                vh = kv_ref[1, b * HQ_LOCAL + h]
                scores = lax.dot_general(
                    qh, kh, (((1,), (1,)), ((), ())),
                    preferred_element_type=f32)
                w = jnp.exp(scores)
                s = jnp.sum(w, axis=-1, keepdims=True)
                ctx = jnp.dot(w.astype(bf16), vh,
                              preferred_element_type=f32)
                ctx_ref[b * SQ:(b + 1) * SQ, h * DH:(h + 1) * DH] = (
                    ctx * (1.0 / s)).astype(bf16)

        pl.semaphore_wait(barrier_sem, N_DEV - 1)

        def send_chunk(c):
            @pl.when(my != c)
            def _():
                rdma = pltpu.make_async_remote_copy(
                    src_ref=part_ref.at[pl.ds(c * CHUNK, CHUNK), :],
                    dst_ref=rs_ref.at[my],
                    send_sem=rs_send_sems.at[c],
                    recv_sem=rs_recv_sems.at[my],
                    device_id=(c,),
                    device_id_type=pl.DeviceIdType.MESH,
                )
                rdma.start()

        half = ROWS // 2
        part_ref[:half, :] = jnp.dot(
            ctx_ref[:half, :], wo_ref[...],
            preferred_element_type=f32).astype(bf16)
        for c in range(N_DEV // 2):
            send_chunk(c)
        part_ref[half:, :] = jnp.dot(
            ctx_ref[half:, :], wo_ref[...],
            preferred_element_type=f32).astype(bf16)
        for c in range(N_DEV // 2, N_DEV):
            send_chunk(c)

        red_ref[...] = part_ref[pl.ds(my * CHUNK, CHUNK), :].astype(f32)
        for s_ in range(N_DEV):
            @pl.when(my != s_)
            def _():
                recv = pltpu.make_async_remote_copy(
                    src_ref=rs_ref.at[s_], dst_ref=rs_ref.at[s_],
                    send_sem=rs_send_sems.at[s_],
                    recv_sem=rs_recv_sems.at[s_],
                    device_id=(s_,), device_id_type=pl.DeviceIdType.MESH,
                )
                recv.wait_recv()
                red_ref[...] = red_ref[...] + rs_ref[s_].astype(f32)

        red16_ref[...] = red_ref[...].astype(bf16)
        out_ref[pl.ds(my * CHUNK, CHUNK), :] = red_ref[...]
        for c in range(N_DEV):
            @pl.when(my != c)
            def _():
                rdma = pltpu.make_async_remote_copy(
                    src_ref=red16_ref,
                    dst_ref=ag_ref.at[my],
                    send_sem=ag_send_sems.at[c],
                    recv_sem=ag_recv_sems.at[my],
                    device_id=(c,),
                    device_id_type=pl.DeviceIdType.MESH,
                )
                rdma.start()
        for s_ in range(N_DEV):
            @pl.when(my != s_)
            def _():
                recv = pltpu.make_async_remote_copy(
                    src_ref=red16_ref,
                    dst_ref=ag_ref.at[s_],
                    send_sem=ag_send_sems.at[s_],
                    recv_sem=ag_recv_sems.at[s_],
                    device_id=(s_,), device_id_type=pl.DeviceIdType.MESH,
                )
                recv.wait_recv()
                out_ref[pl.ds(s_ * CHUNK, CHUNK), :] = ag_ref[s_].astype(f32)

        for c in range(N_DEV):
            @pl.when(my != c)
            def _():
                send = pltpu.make_async_remote_copy(
                    src_ref=part_ref.at[pl.ds(c * CHUNK, CHUNK), :],
                    dst_ref=rs_ref.at[my],
                    send_sem=rs_send_sems.at[c],
                    recv_sem=rs_recv_sems.at[my],
                    device_id=(c,), device_id_type=pl.DeviceIdType.MESH,
                )
                send.wait_send()
                send2 = pltpu.make_async_remote_copy(
                    src_ref=red16_ref,
                    dst_ref=ag_ref.at[my],
                    send_sem=ag_send_sems.at[c],
                    recv_sem=ag_recv_sems.at[my],
                    device_id=(c,), device_id_type=pl.DeviceIdType.MESH,
                )
                send2.wait_send()

    my = lax.axis_index("i")
    wq_local = lax.dynamic_slice(
        Wq, (0, my * HD_LOCAL), (Wq.shape[0], HD_LOCAL)).astype(bf16)
    wo_local = lax.dynamic_slice(
        Wo, (my * HD_LOCAL, 0), (HD_LOCAL, Wo.shape[1])).astype(bf16)
    kv2 = (jnp.stack([K_ext, V_ext])
           .transpose(0, 1, 3, 2, 4)
           .reshape(2, B * HQ_LOCAL, SQ, DH)
           .astype(bf16))
    x2 = x.reshape(ROWS, -1)

    out2d = pl.pallas_call(
        body,
        out_shape=jax.ShapeDtypeStruct((ROWS, D_MODEL), f32),
        in_specs=[pl.BlockSpec(memory_space=pltpu.VMEM)] * 4,
        out_specs=pl.BlockSpec(memory_space=pltpu.VMEM),
        scratch_shapes=[
            pltpu.VMEM((ROWS, HD_LOCAL), bf16),
            pltpu.VMEM((ROWS, D_MODEL), bf16),
            pltpu.VMEM((CHUNK, D_MODEL), f32),
            pltpu.VMEM((CHUNK, D_MODEL), bf16),
            pltpu.VMEM((N_DEV, CHUNK, D_MODEL), bf16),
            pltpu.VMEM((N_DEV, CHUNK, D_MODEL), bf16),
            pltpu.SemaphoreType.DMA((N_DEV,)),
            pltpu.SemaphoreType.DMA((N_DEV,)),
            pltpu.SemaphoreType.DMA((N_DEV,)),
            pltpu.SemaphoreType.DMA((N_DEV,)),
        ],
        compiler_params=pltpu.CompilerParams(collective_id=0),
    )(x2, wq_local, kv2, wo_local)
    return out2d.reshape(B, SQ, D_MODEL)


# device time: 16537 ns/iter; 1.2199x vs baseline; 1.0301x over previous
import jax
import jax.numpy as jnp
from jax import lax
from jax.experimental import pallas as pl
from jax.experimental.pallas import tpu as pltpu

N_DEV = 8
B = 2
SQ = 128
HQ_LOCAL = 4
DH = 64
HD_LOCAL = HQ_LOCAL * DH
D_MODEL = 512
ROWS = B * SQ
CHUNK = ROWS // N_DEV

bf16 = jnp.bfloat16
f32 = jnp.float32


def kernel(x, Wq, K_ext, V_ext, Wo):
    def body(x_ref, wq_ref, kv_ref, wo_ref, out_ref,
             ctx_ref, part_ref, red_ref, red16_ref, rs_ref, ag_ref,
             rs_send_sems, rs_recv_sems, ag_send_sems, ag_recv_sems):
        my = lax.axis_index("i")

        barrier_sem = pltpu.get_barrier_semaphore()
        for o in range(1, N_DEV):
            pl.semaphore_signal(barrier_sem, inc=1,
                                device_id=(lax.rem(my + o, N_DEV),),
                                device_id_type=pl.DeviceIdType.MESH)

        q = jnp.dot(x_ref[...].astype(bf16), wq_ref[...],
                    preferred_element_type=f32)
        q = (q * 0.125).astype(bf16)
        for b in range(B):
            for h in range(HQ_LOCAL):
                qh = q[b * SQ:(b + 1) * SQ, h * DH:(h + 1) * DH]
                kh = kv_ref[0, b * HQ_LOCAL + h]
                vh = kv_ref[1, b * HQ_LOCAL + h]
                scores = lax.dot_general(
                    qh, kh, (((1,), (1,)), ((), ())),
                    preferred_element_type=f32)
                w = jnp.exp(scores)
                s = jnp.sum(w, axis=-1, keepdims=True)
                ctx = jnp.dot(w.astype(bf16), vh,
                              preferred_element_type=f32)
                ctx_ref[b * SQ:(b + 1) * SQ, h * DH:(h + 1) * DH] = (
                    ctx * (1.0 / s)).astype(bf16)

        pl.semaphore_wait(barrier_sem, N_DEV - 1)

        def rs_send(o, first_half):
            p = lax.rem(my + o, N_DEV)
            @pl.when((p < N_DEV // 2) == first_half)
            def _():
                rdma = pltpu.make_async_remote_copy(
                    src_ref=part_ref.at[pl.ds(p * CHUNK, CHUNK), :],
                    dst_ref=rs_ref.at[o],
                    send_sem=rs_send_sems.at[o],
                    recv_sem=rs_recv_sems.at[o],
                    device_id=(p,),
                    device_id_type=pl.DeviceIdType.MESH,
                )
                rdma.start()

        half = ROWS // 2
        part_ref[:half, :] = jnp.dot(
            ctx_ref[:half, :], wo_ref[...],
            preferred_element_type=f32).astype(bf16)
        for o in range(1, N_DEV):
            rs_send(o, True)
        part_ref[half:, :] = jnp.dot(
            ctx_ref[half:, :], wo_ref[...],
            preferred_element_type=f32).astype(bf16)
        for o in range(1, N_DEV):
            rs_send(o, False)

        red = part_ref[pl.ds(my * CHUNK, CHUNK), :].astype(f32)
        for o in range(1, N_DEV):
            recv = pltpu.make_async_remote_copy(
                src_ref=rs_ref.at[o], dst_ref=rs_ref.at[o],
                send_sem=rs_send_sems.at[o], recv_sem=rs_recv_sems.at[o],
                device_id=(my,), device_id_type=pl.DeviceIdType.MESH,
            )
            recv.wait_recv()
            red = red + rs_ref[o].astype(f32)
        red_ref[...] = red
        red16_ref[...] = red.astype(bf16)
        out_ref[pl.ds(my * CHUNK, CHUNK), :] = red_ref[...]

        for o in range(1, N_DEV):
            rdma = pltpu.make_async_remote_copy(
                src_ref=red16_ref,
                dst_ref=ag_ref.at[o],
                send_sem=ag_send_sems.at[o],
                recv_sem=ag_recv_sems.at[o],
                device_id=(lax.rem(my + o, N_DEV),),
                device_id_type=pl.DeviceIdType.MESH,
            )
            rdma.start()
        for o in range(1, N_DEV):
            recv = pltpu.make_async_remote_copy(
                src_ref=red16_ref, dst_ref=ag_ref.at[o],
                send_sem=ag_send_sems.at[o], recv_sem=ag_recv_sems.at[o],
                device_id=(my,), device_id_type=pl.DeviceIdType.MESH,
            )
            recv.wait_recv()
            sender = lax.rem(my - o + N_DEV, N_DEV)
            out_ref[pl.ds(sender * CHUNK, CHUNK), :] = ag_ref[o].astype(f32)

        for o in range(1, N_DEV):
            p = lax.rem(my + o, N_DEV)
            send = pltpu.make_async_remote_copy(
                src_ref=part_ref.at[pl.ds(p * CHUNK, CHUNK), :],
                dst_ref=rs_ref.at[o],
                send_sem=rs_send_sems.at[o], recv_sem=rs_recv_sems.at[o],
                device_id=(p,), device_id_type=pl.DeviceIdType.MESH,
            )
            send.wait_send()
            send2 = pltpu.make_async_remote_copy(
                src_ref=red16_ref, dst_ref=ag_ref.at[o],
                send_sem=ag_send_sems.at[o], recv_sem=ag_recv_sems.at[o],
                device_id=(p,), device_id_type=pl.DeviceIdType.MESH,
            )
            send2.wait_send()

    my = lax.axis_index("i")
    wq_local = lax.dynamic_slice(
        Wq, (0, my * HD_LOCAL), (Wq.shape[0], HD_LOCAL)).astype(bf16)
    wo_local = lax.dynamic_slice(
        Wo, (my * HD_LOCAL, 0), (HD_LOCAL, Wo.shape[1])).astype(bf16)
    kv2 = (jnp.stack([K_ext, V_ext])
           .transpose(0, 1, 3, 2, 4)
           .reshape(2, B * HQ_LOCAL, SQ, DH)
           .astype(bf16))
    x2 = x.reshape(ROWS, -1)

    out2d = pl.pallas_call(
        body,
        out_shape=jax.ShapeDtypeStruct((ROWS, D_MODEL), f32),
        in_specs=[pl.BlockSpec(memory_space=pltpu.VMEM)] * 4,
        out_specs=pl.BlockSpec(memory_space=pltpu.VMEM),
        scratch_shapes=[
            pltpu.VMEM((ROWS, HD_LOCAL), bf16),
            pltpu.VMEM((ROWS, D_MODEL), bf16),
            pltpu.VMEM((CHUNK, D_MODEL), f32),
            pltpu.VMEM((CHUNK, D_MODEL), bf16),
            pltpu.VMEM((N_DEV, CHUNK, D_MODEL), bf16),
            pltpu.VMEM((N_DEV, CHUNK, D_MODEL), bf16),
            pltpu.SemaphoreType.DMA((N_DEV,)),
            pltpu.SemaphoreType.DMA((N_DEV,)),
            pltpu.SemaphoreType.DMA((N_DEV,)),
            pltpu.SemaphoreType.DMA((N_DEV,)),
        ],
        compiler_params=pltpu.CompilerParams(collective_id=0),
    )(x2, wq_local, kv2, wo_local)
    return out2d.reshape(B, SQ, D_MODEL)
